# speedup vs baseline: 5.0886x; 5.0886x over previous
import functools

import jax
import jax.numpy as jnp
from jax import lax
from jax.experimental import pallas as pl
from jax.experimental.pallas import tpu as pltpu
from jax.experimental.pallas import tpu_sc as plsc

N = 10000
E = 320000
D = 128
NPAD = 10240
NC = 2
NS = 16
NW = NC * NS
CH = 128
CPW = -(-E // (NW * CH))
TOTCH = NW * CPW
EPAD = TOTCH * CH
RPT = NPAD // NS

_mesh = plsc.VectorSubcoreMesh(core_axis_name="c", subcore_axis_name="s")


@functools.partial(
    pl.kernel,
    out_type=(
        jax.ShapeDtypeStruct((NC * NPAD,), jnp.float32),
        jax.ShapeDtypeStruct((NC * NPAD,), jnp.float32),
    ),
    mesh=_mesh,
    scratch_types=[
        pltpu.VMEM((CH,), jnp.int32),
        pltpu.VMEM((CH,), jnp.int32),
        pltpu.VMEM((CH,), jnp.float32),
        pltpu.VMEM((RPT,), jnp.float32),
        pltpu.VMEM_SHARED((NPAD,), jnp.float32),
        pltpu.VMEM_SHARED((NPAD,), jnp.float32),
    ],
)
def _deg_kernel(src_hbm, dst_hbm, dego_hbm, degi_hbm,
                src_v, dst_v, ones_v, z_v, dego_sh, degi_sh):
    cid = lax.axis_index("c")
    sid = lax.axis_index("s")
    wid = sid * NC + cid

    for j in range(CH // 16):
        ones_v[pl.ds(j * 16, 16)] = jnp.full((16,), 1.0, jnp.float32)

    def zb(i, _):
        z_v[pl.ds(i * 16, 16)] = jnp.zeros((16,), jnp.float32)
        return ()
    lax.fori_loop(0, RPT // 16, zb, ())
    pltpu.sync_copy(z_v, dego_sh.at[pl.ds(sid * RPT, RPT)])
    pltpu.sync_copy(z_v, degi_sh.at[pl.ds(sid * RPT, RPT)])
    plsc.subcore_barrier()

    def body(i, _):
        chunk = wid * CPW + i
        pltpu.sync_copy(src_hbm.at[chunk], src_v)
        pltpu.sync_copy(dst_hbm.at[chunk], dst_v)
        pltpu.sync_copy(ones_v, dego_sh.at[src_v], add=True)
        pltpu.sync_copy(ones_v, degi_sh.at[dst_v], add=True)
        return ()
    lax.fori_loop(0, CPW, body, ())
    plsc.subcore_barrier()

    off = cid * NPAD + sid * RPT
    pltpu.sync_copy(dego_sh.at[pl.ds(sid * RPT, RPT)], dego_hbm.at[pl.ds(off, RPT)])
    pltpu.sync_copy(degi_sh.at[pl.ds(sid * RPT, RPT)], degi_hbm.at[pl.ds(off, RPT)])


@functools.partial(
    pl.kernel,
    out_type=jax.ShapeDtypeStruct((NC * NPAD, D), jnp.float32),
    mesh=_mesh,
    scratch_types=[
        pltpu.VMEM((CH,), jnp.int32),
        pltpu.VMEM((CH,), jnp.int32),
        pltpu.VMEM((CH, D), jnp.float32),
        pltpu.VMEM_SHARED((NPAD, D), jnp.float32),
    ],
)
def _agg_kernel(src_hbm, dst_hbm, hs_hbm, out_hbm,
                src_v, dst_v, rows_v, acc_sh):
    cid = lax.axis_index("c")
    sid = lax.axis_index("s")
    wid = sid * NC + cid

    def zr(r, _):
        def zc(j, _):
            rows_v[r, pl.ds(j * 16, 16)] = jnp.zeros((16,), jnp.float32)
            return ()
        lax.fori_loop(0, D // 16, zc, ())
        return ()
    lax.fori_loop(0, CH, zr, ())

    def zcopy(i, _):
        pltpu.sync_copy(rows_v, acc_sh.at[pl.ds(sid * RPT + i * CH, CH)])
        return ()
    lax.fori_loop(0, RPT // CH, zcopy, ())
    plsc.subcore_barrier()

    def body(i, _):
        chunk = wid * CPW + i
        pltpu.sync_copy(src_hbm.at[chunk], src_v)
        pltpu.sync_copy(dst_hbm.at[chunk], dst_v)
        pltpu.sync_copy(hs_hbm.at[src_v], rows_v)
        pltpu.sync_copy(rows_v, acc_sh.at[dst_v], add=True)
        return ()
    lax.fori_loop(0, CPW, body, ())
    plsc.subcore_barrier()

    off = cid * NPAD + sid * RPT
    pltpu.sync_copy(acc_sh.at[pl.ds(sid * RPT, RPT)], out_hbm.at[pl.ds(off, RPT)])


_RB = 256
_GRID = NPAD // _RB


def _prep_body(f_ref, do0_ref, do1_ref, di0_ref, di1_ref,
               hs_ref, nsrc_ref, ndst_ref):
    do = do0_ref[...] + do1_ref[...] + 1.0
    di = di0_ref[...] + di1_ref[...] + 1.0
    ns = lax.rsqrt(do)
    nd = lax.rsqrt(di)
    nsrc_ref[...] = ns
    ndst_ref[...] = nd
    hs_ref[...] = f_ref[...] * ns


def _layer1_body(m0_ref, m1_ref, hs_ref, nd_ref, ns_ref, w_ref, b_ref, out_ref):
    m = (m0_ref[...] + m1_ref[...] + hs_ref[...]) * nd_ref[...]
    h = jnp.dot(m, w_ref[...], preferred_element_type=jnp.float32) + b_ref[...]
    out_ref[...] = jnp.maximum(h, 0.0) * ns_ref[...]


def _layer2_body(m0_ref, m1_ref, hs_ref, nd_ref, w_ref, b_ref, wp_ref, bp_ref,
                 out_ref):
    m = (m0_ref[...] + m1_ref[...] + hs_ref[...]) * nd_ref[...]
    h = jnp.dot(m, w_ref[...], preferred_element_type=jnp.float32) + b_ref[...]
    h = jnp.maximum(h, 0.0)
    out_ref[...] = jnp.sum(h * wp_ref[...], axis=1, keepdims=True) + bp_ref[...]


def _row_spec():
    return pl.BlockSpec((_RB, D), lambda i: (i, 0))


def _col_spec():
    return pl.BlockSpec((_RB, 1), lambda i: (i, 0))


def _full_spec(shape):
    return pl.BlockSpec(shape, lambda i: (0, 0))


_prep_call = pl.pallas_call(
    _prep_body,
    grid=(_GRID,),
    in_specs=[_row_spec(), _col_spec(), _col_spec(), _col_spec(), _col_spec()],
    out_specs=[_row_spec(), _col_spec(), _col_spec()],
    out_shape=[
        jax.ShapeDtypeStruct((NPAD, D), jnp.float32),
        jax.ShapeDtypeStruct((NPAD, 1), jnp.float32),
        jax.ShapeDtypeStruct((NPAD, 1), jnp.float32),
    ],
)

_layer1_call = pl.pallas_call(
    _layer1_body,
    grid=(_GRID,),
    in_specs=[_row_spec(), _row_spec(), _row_spec(), _col_spec(), _col_spec(),
              _full_spec((D, D)), _full_spec((1, D))],
    out_specs=_row_spec(),
    out_shape=jax.ShapeDtypeStruct((NPAD, D), jnp.float32),
)

_layer2_call = pl.pallas_call(
    _layer2_body,
    grid=(_GRID,),
    in_specs=[_row_spec(), _row_spec(), _row_spec(), _col_spec(),
              _full_spec((D, D)), _full_spec((1, D)), _full_spec((1, D)),
              _full_spec((1, 1))],
    out_specs=_col_spec(),
    out_shape=jax.ShapeDtypeStruct((NPAD, 1), jnp.float32),
)


def kernel(features, edge_index, W1, b1, W2, b2, Wp, bp):
    src = edge_index[0].astype(jnp.int32)
    dst = edge_index[1].astype(jnp.int32)
    padv = jnp.full((EPAD - E,), N, dtype=jnp.int32)
    src2d = jnp.concatenate([src, padv]).reshape(TOTCH, CH)
    dst2d = jnp.concatenate([dst, padv]).reshape(TOTCH, CH)

    dego_p, degi_p = _deg_kernel(src2d, dst2d)
    do0 = dego_p[:NPAD].reshape(NPAD, 1)
    do1 = dego_p[NPAD:].reshape(NPAD, 1)
    di0 = degi_p[:NPAD].reshape(NPAD, 1)
    di1 = degi_p[NPAD:].reshape(NPAD, 1)

    fpad = jnp.pad(features, ((0, NPAD - N), (0, 0)))
    hs0, nsrc, ndst = _prep_call(fpad, do0, do1, di0, di1)

    m1 = _agg_kernel(src2d, dst2d, hs0)
    h1s = _layer1_call(m1[:NPAD], m1[NPAD:], hs0, ndst, nsrc,
                       W1, b1.reshape(1, D))
    m2 = _agg_kernel(src2d, dst2d, h1s)
    logits = _layer2_call(m2[:NPAD], m2[NPAD:], h1s, ndst,
                          W2, b2.reshape(1, D), Wp.reshape(1, D),
                          bp.reshape(1, 1))
    return logits[:N]
